# baseline (speedup 1.0000x reference)
import functools

import jax
import jax.numpy as jnp
from jax import lax
from jax.experimental import pallas as pl
from jax.experimental.pallas import tpu as pltpu
from jax.experimental.pallas import tpu_sc as plsc

_NUM_CORES = 2
_NUM_SUBCORES = 16
_NW = _NUM_CORES * _NUM_SUBCORES
_NB = 4
_NBUF = 4


@functools.lru_cache(maxsize=None)
def _build(batch, seq, D):
    b_per_w = batch // _NW
    n_groups = b_per_w // _NB
    n_outer = n_groups // _NBUF
    mesh = plsc.VectorSubcoreMesh(
        core_axis_name="c", subcore_axis_name="s",
        num_cores=_NUM_CORES, num_subcores=_NUM_SUBCORES)

    @functools.partial(
        pl.kernel,
        out_type=jax.ShapeDtypeStruct((batch, seq, D), jnp.float32),
        mesh=mesh,
        scratch_types=[
            pltpu.VMEM((b_per_w, seq), jnp.int32),
            pltpu.VMEM((_NBUF, _NB, seq, D), jnp.float32),
        ] + [pltpu.SemaphoreType.DMA] * (2 * _NBUF),
        compiler_params=pltpu.CompilerParams(use_tc_tiling_on_sc=False),
    )
    def gather_kernel(idx_hbm, table_hbm, out_hbm, idx_v, rows_v, *sems):
        gsems, osems = sems[:_NBUF], sems[_NBUF:]
        wid = lax.axis_index("s") * _NUM_CORES + lax.axis_index("c")
        b0 = wid * b_per_w
        pltpu.sync_copy(idx_hbm.at[pl.ds(b0, b_per_w)], idx_v)

        def g_copy(g, k, buf):
            return pltpu.make_async_copy(
                table_hbm.at[idx_v.at[g * _NB + k]], rows_v.at[buf, k],
                gsems[buf])

        def o_copy(g, buf):
            return pltpu.make_async_copy(
                rows_v.at[buf], out_hbm.at[pl.ds(b0 + g * _NB, _NB)],
                osems[buf])

        for buf in range(_NBUF):
            for k in range(_NB):
                g_copy(buf, k, buf).start()

        def body(go, carry):
            for buf in range(_NBUF):
                g = go * _NBUF + buf
                for k in range(_NB):
                    g_copy(g, k, buf).wait()
                o_copy(g, buf).start()
            for buf in range(_NBUF):
                g = go * _NBUF + buf
                o_copy(g, buf).wait()
                for k in range(_NB):
                    g_copy(g + _NBUF, k, buf).start()
            return carry

        lax.fori_loop(0, n_outer - 1, body, 0)

        go = n_outer - 1
        for buf in range(_NBUF):
            g = go * _NBUF + buf
            for k in range(_NB):
                g_copy(g, k, buf).wait()
            o_copy(g, buf).start()
        for buf in range(_NBUF):
            o_copy(go * _NBUF + buf, buf).wait()

    return gather_kernel


def kernel(x, weight):
    batch, seq = x.shape
    D = weight.shape[1]
    return _build(batch, seq, D)(x, weight)


# speedup vs baseline: 1.2372x; 1.2372x over previous
import functools

import jax
import jax.experimental.layout
import jax.numpy as jnp
from jax import lax
from jax.experimental import pallas as pl
from jax.experimental.pallas import tpu as pltpu
from jax.experimental.pallas import tpu_sc as plsc

_NUM_CORES = 2
_NUM_SUBCORES = 16
_NW = _NUM_CORES * _NUM_SUBCORES
_NB = 4
_NBUF = 4


@functools.lru_cache(maxsize=None)
def _build(batch, seq, D):
    b_per_w = batch // _NW
    n_groups = b_per_w // _NB
    n_outer = n_groups // _NBUF
    mesh = plsc.VectorSubcoreMesh(
        core_axis_name="c", subcore_axis_name="s",
        num_cores=_NUM_CORES, num_subcores=_NUM_SUBCORES)

    @functools.partial(
        pl.kernel,
        out_type=jax.ShapeDtypeStruct((batch, seq, D), jnp.float32),
        mesh=mesh,
        scratch_types=[
            pltpu.VMEM((b_per_w, seq), jnp.int32),
            pltpu.VMEM((_NBUF, _NB, seq, D), jnp.float32),
        ] + [pltpu.SemaphoreType.DMA] * (2 * _NBUF),
        compiler_params=pltpu.CompilerParams(use_tc_tiling_on_sc=False),
    )
    def gather_kernel(idx_hbm, table_hbm, out_hbm, idx_v, rows_v, *sems):
        gsems, osems = sems[:_NBUF], sems[_NBUF:]
        wid = lax.axis_index("s") * _NUM_CORES + lax.axis_index("c")
        b0 = wid * b_per_w
        pltpu.sync_copy(idx_hbm.at[pl.ds(b0, b_per_w)], idx_v)

        def g_copy(g, k, buf):
            return pltpu.make_async_copy(
                table_hbm.at[idx_v.at[g * _NB + k]], rows_v.at[buf, k],
                gsems[buf])

        def o_copy(g, buf):
            return pltpu.make_async_copy(
                rows_v.at[buf], out_hbm.at[pl.ds(b0 + g * _NB, _NB)],
                osems[buf])

        for buf in range(_NBUF):
            for k in range(_NB):
                g_copy(buf, k, buf).start()

        def body(go, carry):
            for buf in range(_NBUF):
                g = go * _NBUF + buf
                for k in range(_NB):
                    g_copy(g, k, buf).wait()
                o_copy(g, buf).start()
            for buf in range(_NBUF):
                g = go * _NBUF + buf
                o_copy(g, buf).wait()
                for k in range(_NB):
                    g_copy(g + _NBUF, k, buf).start()
            return carry

        lax.fori_loop(0, n_outer - 1, body, 0)

        go = n_outer - 1
        for buf in range(_NBUF):
            g = go * _NBUF + buf
            for k in range(_NB):
                g_copy(g, k, buf).wait()
            o_copy(g, buf).start()
        for buf in range(_NBUF):
            o_copy(go * _NBUF + buf, buf).wait()

    return gather_kernel


def kernel(x, weight):
    batch, seq = x.shape
    D = weight.shape[1]
    wt = jax.experimental.layout.with_layout_constraint(
        weight,
        jax.experimental.layout.Layout(
            major_to_minor=(0, 1), tiling=()))
    return _build(batch, seq, D)(x + x, wt)


# speedup vs baseline: 1.2431x; 1.0048x over previous
import functools

import jax
import jax.experimental.layout
import jax.numpy as jnp
from jax import lax
from jax.experimental import pallas as pl
from jax.experimental.pallas import tpu as pltpu
from jax.experimental.pallas import tpu_sc as plsc

_NUM_CORES = 2
_NUM_SUBCORES = 16
_NW = _NUM_CORES * _NUM_SUBCORES
_NB = 4
_NBUF = 4


@functools.lru_cache(maxsize=None)
def _build(batch, seq, D):
    b_per_w = batch // _NW
    n_groups = b_per_w // _NB
    n_outer = n_groups // _NBUF
    mesh = plsc.VectorSubcoreMesh(
        core_axis_name="c", subcore_axis_name="s",
        num_cores=_NUM_CORES, num_subcores=_NUM_SUBCORES)

    @functools.partial(
        pl.kernel,
        out_type=jax.ShapeDtypeStruct((batch, seq, D), jnp.float32),
        mesh=mesh,
        scratch_types=[
            pltpu.VMEM((b_per_w, seq), jnp.int32),
            pltpu.VMEM((_NBUF, _NB, seq, D), jnp.float32),
        ] + [pltpu.SemaphoreType.DMA] * (2 * _NBUF),
        compiler_params=pltpu.CompilerParams(use_tc_tiling_on_sc=False),
    )
    def gather_kernel(idx_hbm, table_hbm, out_hbm, idx_v, rows_v, *sems):
        gsems, osems = sems[:_NBUF], sems[_NBUF:]
        wid = lax.axis_index("s") * _NUM_CORES + lax.axis_index("c")
        b0 = wid * b_per_w
        pltpu.sync_copy(idx_hbm.at[pl.ds(b0, b_per_w)], idx_v)

        def g_copy(g, k, buf):
            return pltpu.make_async_copy(
                table_hbm.at[idx_v.at[g * _NB + k]], rows_v.at[buf, k],
                gsems[buf])

        def o_copy(g, buf):
            return pltpu.make_async_copy(
                rows_v.at[buf], out_hbm.at[pl.ds(b0 + g * _NB, _NB)],
                osems[buf])

        for buf in range(_NBUF):
            for k in range(_NB):
                g_copy(buf, k, buf).start()

        def body(go, carry):
            for buf in range(_NBUF):
                g = go * _NBUF + buf
                for k in range(_NB):
                    g_copy(g, k, buf).wait()
                o_copy(g, buf).start()
            for buf in range(_NBUF):
                g = go * _NBUF + buf
                o_copy(g, buf).wait()
                for k in range(_NB):
                    g_copy(g + _NBUF, k, buf).start()
            return carry

        lax.fori_loop(0, n_outer - 1, body, 0)

        go = n_outer - 1
        for buf in range(_NBUF):
            g = go * _NBUF + buf
            for k in range(_NB):
                g_copy(g, k, buf).wait()
            o_copy(g, buf).start()
        for buf in range(_NBUF):
            o_copy(go * _NBUF + buf, buf).wait()

    return gather_kernel


def kernel(x, weight):
    batch, seq = x.shape
    D = weight.shape[1]
    wt = jax.experimental.layout.with_layout_constraint(
        weight,
        jax.experimental.layout.Layout(
            major_to_minor=(0, 1), tiling=((8, 128),)))
    return _build(batch, seq, D)(x + x, wt)
